# baseline (speedup 1.0000x reference)
import functools

import jax
import jax.numpy as jnp
from jax import lax
from jax.experimental import pallas as pl
from jax.experimental.pallas import tpu as pltpu
from jax.experimental.pallas import tpu_sc as plsc

B = 16384
F = 128
H = 32
E = 16
NC, NS, L = 2, 16, 16
NW = NC * NS
BPW = B // NW
MLP_BB = 2048


def _mlp_body(uf_ref, w1_ref, b1_ref, w2_ref, b2_ref, out_ref):
    h = jnp.dot(uf_ref[...], w1_ref[...], preferred_element_type=jnp.float32)
    h = jnp.maximum(h + b1_ref[...], 0.0)
    out_ref[...] = (
        jnp.dot(h, w2_ref[...], preferred_element_type=jnp.float32) + b2_ref[...]
    )


_mlp = pl.pallas_call(
    _mlp_body,
    grid=(B // MLP_BB,),
    in_specs=[
        pl.BlockSpec((MLP_BB, F), lambda i: (i, 0)),
        pl.BlockSpec((F, H), lambda i: (0, 0)),
        pl.BlockSpec((1, H), lambda i: (0, 0)),
        pl.BlockSpec((H, E), lambda i: (0, 0)),
        pl.BlockSpec((1, E), lambda i: (0, 0)),
    ],
    out_specs=pl.BlockSpec((MLP_BB, E), lambda i: (i, 0)),
    out_shape=jax.ShapeDtypeStruct((B, E), jnp.float32),
)


@functools.cache
def _make_sc_score():
    mesh = plsc.VectorSubcoreMesh(
        core_axis_name="c", subcore_axis_name="s", num_cores=NC, num_subcores=NS
    )

    @functools.partial(
        pl.kernel,
        out_type=jax.ShapeDtypeStruct((B,), jnp.float32),
        mesh=mesh,
        compiler_params=pltpu.CompilerParams(
            needs_layout_passes=False, use_tc_tiling_on_sc=False
        ),
        scratch_types=[
            pltpu.VMEM((BPW,), jnp.int32),
            pltpu.VMEM((BPW,), jnp.int32),
            pltpu.VMEM((BPW, E), jnp.float32),
            pltpu.VMEM((BPW, E), jnp.float32),
            pltpu.VMEM((BPW, E), jnp.float32),
            pltpu.VMEM((BPW,), jnp.float32),
            pltpu.SemaphoreType.DMA,
            pltpu.SemaphoreType.DMA,
        ],
    )
    def _sc_score(uid_hbm, iid_hbm, content_hbm, utab_hbm, itab_hbm, out_hbm,
                  uidx_v, iidx_v, urows_v, irows_v, c_v, s_v, usem, isem):
        wid = lax.axis_index("s") * NC + lax.axis_index("c")
        base = wid * BPW
        pltpu.sync_copy(uid_hbm.at[pl.ds(base, BPW)], uidx_v)
        pltpu.sync_copy(iid_hbm.at[pl.ds(base, BPW)], iidx_v)
        cu = pltpu.async_copy(utab_hbm.at[uidx_v], urows_v, usem)
        ci = pltpu.async_copy(itab_hbm.at[iidx_v], irows_v, isem)
        pltpu.sync_copy(content_hbm.at[pl.ds(base, BPW), :], c_v)
        cu.wait()
        ci.wait()

        lanes = lax.iota(jnp.int32, L)

        def blk_body(blk, carry):
            acc = jnp.zeros((L,), jnp.float32)
            for j in range(L):
                b = blk * L + j
                p = (urows_v[b, :] + c_v[b, :]) * irows_v[b, :]
                score = jnp.sum(p, axis=0)
                acc = jnp.where(lanes == j, score, acc)
            s_v[pl.ds(blk * L, L)] = acc
            return carry

        lax.fori_loop(0, BPW // L, blk_body, 0)
        pltpu.sync_copy(s_v, out_hbm.at[pl.ds(base, BPW)])

    return _sc_score


def kernel(user_ids, item_ids, user_features, cf_user_table, cf_item_table,
           W1, b1, W2, b2):
    content = _mlp(user_features, W1, b1.reshape(1, H), W2, b2.reshape(1, E))
    return _make_sc_score()(user_ids, item_ids, content,
                            cf_user_table, cf_item_table)
